# baseline (speedup 1.0000x reference)
import jax
import jax.numpy as jnp
from jax import lax
from jax.experimental import pallas as pl
from jax.experimental.pallas import tpu as pltpu
from jax.experimental.pallas import tpu_sc as plsc

N = 10000
D = 128
E = 320000
EPS = 1e-5

NC = 2
NS = 16
NW = NC * NS
EW = E // NW
CH = 100
NCH = EW // CH
WIN = 4
NWIN = NCH // WIN
CHD = 125
NCHD = EW // CHD
RPT = 640
RPT_LAST = N - 15 * RPT
ZB = 80
DEGW = 128

_sc_mesh = plsc.VectorSubcoreMesh(
    core_axis_name="c", subcore_axis_name="s", num_cores=NC, num_subcores=NS)


def _zero_rows(zeros_hbm, acc_sh, base, nrows):
    for k in range(nrows // ZB):
        pltpu.sync_copy(zeros_hbm, acc_sh.at[pl.ds(base + k * ZB, ZB)])


def _drain_rows(acc_sh, out, c, base, nrows):
    pltpu.sync_copy(acc_sh.at[pl.ds(base, nrows)], out.at[c, pl.ds(base, nrows)])


def _deg_body(dst3d, ones_hbm, zeros_hbm, out, dst_v, ones_v, acc_sh):
    c = lax.axis_index("c")
    s = lax.axis_index("s")
    wid = s * NC + c
    pltpu.sync_copy(dst3d.at[wid], dst_v)
    pltpu.sync_copy(ones_hbm, ones_v)
    base = s * RPT

    @pl.when(s < NS - 1)
    def _():
        _zero_rows(zeros_hbm, acc_sh, base, RPT)

    @pl.when(s == NS - 1)
    def _():
        _zero_rows(zeros_hbm, acc_sh, base, RPT_LAST)

    plsc.subcore_barrier()

    def body(j, carry):
        pltpu.sync_copy(ones_v, acc_sh.at[dst_v.at[j]], add=True)
        return carry

    lax.fori_loop(0, NCHD, body, 0)
    plsc.subcore_barrier()

    @pl.when(s < NS - 1)
    def _():
        _drain_rows(acc_sh, out, c, base, RPT)

    @pl.when(s == NS - 1)
    def _():
        _drain_rows(acc_sh, out, c, base, RPT_LAST)


_deg_call = pl.kernel(
    _deg_body,
    out_type=jax.ShapeDtypeStruct((NC, N, DEGW), jnp.float32),
    mesh=_sc_mesh,
    scratch_types=[
        pltpu.VMEM((NCHD, CHD), jnp.int32),
        pltpu.VMEM((CHD, DEGW), jnp.float32),
        pltpu.VMEM_SHARED((N, DEGW), jnp.float32),
    ],
)


def _agg_body(hs, esd3d, zeros_hbm, out,
              ewA, ewB, buf0, buf1, buf2, acc_sh, sem0, sem1, sem2, semi):
    c = lax.axis_index("c")
    s = lax.axis_index("s")
    wid = s * NC + c
    base = s * RPT

    pltpu.sync_copy(esd3d.at[wid, pl.ds(0, 2 * WIN)], ewA)

    @pl.when((c == 0) & (s < NS - 1))
    def _():
        pltpu.sync_copy(hs.at[pl.ds(base, RPT)], acc_sh.at[pl.ds(base, RPT)])

    @pl.when((c == 0) & (s == NS - 1))
    def _():
        pltpu.sync_copy(hs.at[pl.ds(base, RPT_LAST)],
                        acc_sh.at[pl.ds(base, RPT_LAST)])

    @pl.when((c != 0) & (s < NS - 1))
    def _():
        _zero_rows(zeros_hbm, acc_sh, base, RPT)

    @pl.when((c != 0) & (s == NS - 1))
    def _():
        _zero_rows(zeros_hbm, acc_sh, base, RPT_LAST)

    plsc.subcore_barrier()

    bufs = (buf0, buf1, buf2)
    sems = (sem0, sem1, sem2)
    eww = (ewA, ewB)

    pltpu.async_copy(hs.at[ewA.at[0]], buf0, sem0)
    pltpu.async_copy(hs.at[ewA.at[2]], buf1, sem1)

    for w in range(NWIN):
        cw = eww[w % 2]
        nw = eww[(w + 1) % 2]
        for k in range(WIN):
            j = w * WIN + k
            b, sm = bufs[j % 3], sems[j % 3]
            nb, nsm = bufs[(j + 2) % 3], sems[(j + 2) % 3]
            pltpu.make_async_copy(hs.at[cw.at[2 * k]], b, sm).wait()
            if k < WIN - 2:
                pltpu.async_copy(hs.at[cw.at[2 * k + 4]], nb, nsm)
            elif w < NWIN - 1:
                if k == WIN - 2:
                    pltpu.make_async_copy(esd3d.at[wid, pl.ds(0, 2 * WIN)], nw,
                                          semi).wait()
                    pltpu.async_copy(hs.at[nw.at[0]], nb, nsm)
                else:
                    pltpu.async_copy(hs.at[nw.at[2]], nb, nsm)
            pltpu.sync_copy(b, acc_sh.at[cw.at[2 * k + 1]], add=True)
            if k == 0 and w < NWIN - 1:
                pltpu.async_copy(
                    esd3d.at[wid, pl.ds((w + 1) * 2 * WIN, 2 * WIN)], nw, semi)

    plsc.subcore_barrier()

    @pl.when(s < NS - 1)
    def _():
        _drain_rows(acc_sh, out, c, base, RPT)

    @pl.when(s == NS - 1)
    def _():
        _drain_rows(acc_sh, out, c, base, RPT_LAST)


_agg_call = pl.kernel(
    _agg_body,
    out_type=jax.ShapeDtypeStruct((NC, N, D), jnp.float32),
    mesh=_sc_mesh,
    scratch_types=[
        pltpu.VMEM((2 * WIN, CH), jnp.int32),
        pltpu.VMEM((2 * WIN, CH), jnp.int32),
        pltpu.VMEM((CH, D), jnp.float32),
        pltpu.VMEM((CH, D), jnp.float32),
        pltpu.VMEM((CH, D), jnp.float32),
        pltpu.VMEM_SHARED((N, D), jnp.float32),
        pltpu.SemaphoreType.DMA,
        pltpu.SemaphoreType.DMA,
        pltpu.SemaphoreType.DMA,
        pltpu.SemaphoreType.DMA,
    ],
)


BLK = 2000
GRID = N // BLK


def _mm0_body(x_ref, w_ref, h_ref):
    h_ref[...] = jnp.dot(x_ref[...], w_ref[...], preferred_element_type=jnp.float32)


def _scale0_body(h_ref, p0_ref, p1_ref, hs_ref, dis_ref):
    deg = p0_ref[:, :1] + p1_ref[:, :1] + 1.0
    dis = lax.rsqrt(deg)
    hs_ref[...] = h_ref[...] * dis
    dis_ref[...] = dis


def _stage0f_body(x_ref, p0_ref, p1_ref, w_ref, hs_ref, dis_ref):
    deg = p0_ref[:, :1] + p1_ref[:, :1] + 1.0
    dis = lax.rsqrt(deg)
    h = jnp.dot(x_ref[...], w_ref[...], preferred_element_type=jnp.float32)
    hs_ref[...] = h * dis
    dis_ref[...] = dis


def _stage_mid_body(a0_ref, a1_ref, dis_ref, b_ref, g_ref, be_ref, m_ref,
                    v_ref, w_ref, hs_ref):
    dis = dis_ref[...]
    y = (a0_ref[...] + a1_ref[...]) * dis + b_ref[...]
    t = (y - m_ref[...]) * lax.rsqrt(v_ref[...] + EPS) * g_ref[...] + be_ref[...]
    t = jnp.maximum(t, 0.0)
    hs_ref[...] = jnp.dot(t, w_ref[...], preferred_element_type=jnp.float32) * dis


def _stage_fin_body(a0_ref, a1_ref, dis_ref, b2_ref, wm1_ref, bm1_ref,
                    wm2_ref, bm2_ref, out_ref):
    y = (a0_ref[...] + a1_ref[...]) * dis_ref[...] + b2_ref[...]
    z = jnp.dot(y, wm1_ref[...], preferred_element_type=jnp.float32) + bm1_ref[...]
    z = jnp.maximum(z, 0.0)
    out_ref[...] = jnp.dot(z, wm2_ref[...], preferred_element_type=jnp.float32) + bm2_ref[...]


def _row_spec(w):
    return pl.BlockSpec((BLK, w), lambda i: (i, 0))


def _full_spec(shape):
    return pl.BlockSpec(shape, lambda i: (0, 0))


_stage0f_call = pl.pallas_call(
    _stage0f_body,
    grid=(GRID,),
    in_specs=[_row_spec(D), _row_spec(DEGW), _row_spec(DEGW), _full_spec((D, D))],
    out_specs=[_row_spec(D), _row_spec(1)],
    out_shape=[
        jax.ShapeDtypeStruct((N, D), jnp.float32),
        jax.ShapeDtypeStruct((N, 1), jnp.float32),
    ],
)

_mm0_call = pl.pallas_call(
    _mm0_body,
    grid=(GRID,),
    in_specs=[_row_spec(D), _full_spec((D, D))],
    out_specs=_row_spec(D),
    out_shape=jax.ShapeDtypeStruct((N, D), jnp.float32),
)

_scale0_call = pl.pallas_call(
    _scale0_body,
    grid=(GRID,),
    in_specs=[_row_spec(D), _row_spec(DEGW), _row_spec(DEGW)],
    out_specs=[_row_spec(D), _row_spec(1)],
    out_shape=[
        jax.ShapeDtypeStruct((N, D), jnp.float32),
        jax.ShapeDtypeStruct((N, 1), jnp.float32),
    ],
)

_stage_mid_call = pl.pallas_call(
    _stage_mid_body,
    grid=(GRID,),
    in_specs=[_row_spec(D), _row_spec(D), _row_spec(1)]
    + [_full_spec((1, D))] * 5 + [_full_spec((D, D))],
    out_specs=_row_spec(D),
    out_shape=jax.ShapeDtypeStruct((N, D), jnp.float32),
)

_stage_fin_call = pl.pallas_call(
    _stage_fin_body,
    grid=(GRID,),
    in_specs=[_row_spec(D), _row_spec(D), _row_spec(1), _full_spec((1, D)),
              _full_spec((D, D)), _full_spec((1, D)), _full_spec((D, D)),
              _full_spec((1, D))],
    out_specs=_row_spec(D),
    out_shape=jax.ShapeDtypeStruct((N, D), jnp.float32),
)


def kernel(x, edge_index, W0, b0, g0, be0, m0, v0, W1, b1, g1, be1, m1, v1,
           W2, b2, Wm1, bm1, Wm2, bm2):
    ei = edge_index.astype(jnp.int32)
    src3d = ei[0].reshape(NW, NCH, CH)
    dst3d = ei[1].reshape(NW, NCH, CH)
    esd3d = jnp.stack([src3d, dst3d], axis=2).reshape(NW, 2 * NCH, CH)
    dst3dd = ei[1].reshape(NW, NCHD, CHD)

    ones16 = jnp.ones((CHD, DEGW), jnp.float32)
    zerosD = jnp.zeros((ZB, D), jnp.float32)

    degp = _deg_call(dst3dd, ones16, zerosD)
    hs0, dis = _stage0f_call(x, degp[0], degp[1], W0)

    acc0 = _agg_call(hs0, esd3d, zerosD)
    hs1 = _stage_mid_call(acc0[0], acc0[1], dis, b0.reshape(1, D),
                          g0.reshape(1, D), be0.reshape(1, D),
                          m0.reshape(1, D), v0.reshape(1, D), W1)

    acc1 = _agg_call(hs1, esd3d, zerosD)
    hs2 = _stage_mid_call(acc1[0], acc1[1], dis, b1.reshape(1, D),
                          g1.reshape(1, D), be1.reshape(1, D),
                          m1.reshape(1, D), v1.reshape(1, D), W2)

    acc2 = _agg_call(hs2, esd3d, zerosD)
    out = _stage_fin_call(acc2[0], acc2[1], dis, b2.reshape(1, D),
                          Wm1, bm1.reshape(1, D), Wm2, bm2.reshape(1, D))
    return out


# speedup vs baseline: 1.0083x; 1.0083x over previous
import jax
import jax.numpy as jnp
from jax import lax
from jax.experimental import pallas as pl
from jax.experimental.pallas import tpu as pltpu
from jax.experimental.pallas import tpu_sc as plsc

N = 10000
D = 128
E = 320000
EPS = 1e-5

NC = 2
NS = 16
NW = NC * NS
EW = E // NW
CH = 100
NCH = EW // CH
WIN = 4
NWIN = NCH // WIN
CHD = 125
NCHD = EW // CHD
RPT = 640
RPT_LAST = N - 15 * RPT
ZB = 80
DEGW = 128

_sc_mesh = plsc.VectorSubcoreMesh(
    core_axis_name="c", subcore_axis_name="s", num_cores=NC, num_subcores=NS)


def _zero_rows(zeros_hbm, acc_sh, base, nrows):
    for k in range(nrows // ZB):
        pltpu.sync_copy(zeros_hbm, acc_sh.at[pl.ds(base + k * ZB, ZB)])


def _drain_rows(acc_sh, out, c, base, nrows):
    pltpu.sync_copy(acc_sh.at[pl.ds(base, nrows)], out.at[c, pl.ds(base, nrows)])


def _deg_body(dst3d, ones_hbm, zeros_hbm, out, dst_v, ones_v, acc_sh):
    c = lax.axis_index("c")
    s = lax.axis_index("s")
    wid = s * NC + c
    pltpu.sync_copy(dst3d.at[wid], dst_v)
    pltpu.sync_copy(ones_hbm, ones_v)
    base = s * RPT

    @pl.when(s < NS - 1)
    def _():
        _zero_rows(zeros_hbm, acc_sh, base, RPT)

    @pl.when(s == NS - 1)
    def _():
        _zero_rows(zeros_hbm, acc_sh, base, RPT_LAST)

    plsc.subcore_barrier()

    def body(j, carry):
        pltpu.sync_copy(ones_v, acc_sh.at[dst_v.at[j]], add=True)
        return carry

    lax.fori_loop(0, NCHD, body, 0)
    plsc.subcore_barrier()

    @pl.when(s < NS - 1)
    def _():
        _drain_rows(acc_sh, out, c, base, RPT)

    @pl.when(s == NS - 1)
    def _():
        _drain_rows(acc_sh, out, c, base, RPT_LAST)


_deg_call = pl.kernel(
    _deg_body,
    out_type=jax.ShapeDtypeStruct((NC, N, DEGW), jnp.float32),
    mesh=_sc_mesh,
    scratch_types=[
        pltpu.VMEM((NCHD, CHD), jnp.int32),
        pltpu.VMEM((CHD, DEGW), jnp.float32),
        pltpu.VMEM_SHARED((N, DEGW), jnp.float32),
    ],
)


def _agg_body(hs, esd3d, zeros_hbm, out,
              ewA, ewB, buf0, buf1, buf2, acc_sh, sem0, sem1, sem2, semi):
    c = lax.axis_index("c")
    s = lax.axis_index("s")
    wid = s * NC + c
    base = s * RPT

    pltpu.sync_copy(esd3d.at[wid, pl.ds(0, 2 * WIN)], ewA)

    @pl.when((c == 0) & (s < NS - 1))
    def _():
        pltpu.sync_copy(hs.at[pl.ds(base, RPT)], acc_sh.at[pl.ds(base, RPT)])

    @pl.when((c == 0) & (s == NS - 1))
    def _():
        pltpu.sync_copy(hs.at[pl.ds(base, RPT_LAST)],
                        acc_sh.at[pl.ds(base, RPT_LAST)])

    @pl.when((c != 0) & (s < NS - 1))
    def _():
        _zero_rows(zeros_hbm, acc_sh, base, RPT)

    @pl.when((c != 0) & (s == NS - 1))
    def _():
        _zero_rows(zeros_hbm, acc_sh, base, RPT_LAST)

    plsc.subcore_barrier()

    bufs = (buf0, buf1, buf2)
    sems = (sem0, sem1, sem2)
    eww = (ewA, ewB)

    pltpu.async_copy(hs.at[ewA.at[0]], buf0, sem0)
    pltpu.async_copy(hs.at[ewA.at[2]], buf1, sem1)

    for w in range(NWIN):
        cw = eww[w % 2]
        nw = eww[(w + 1) % 2]
        for k in range(WIN):
            j = w * WIN + k
            b, sm = bufs[j % 3], sems[j % 3]
            nb, nsm = bufs[(j + 2) % 3], sems[(j + 2) % 3]
            pltpu.make_async_copy(hs.at[cw.at[2 * k]], b, sm).wait()
            if k < WIN - 2:
                pltpu.async_copy(hs.at[cw.at[2 * k + 4]], nb, nsm)
            elif w < NWIN - 1:
                if k == WIN - 2:
                    pltpu.make_async_copy(esd3d.at[wid, pl.ds(0, 2 * WIN)], nw,
                                          semi).wait()
                    pltpu.async_copy(hs.at[nw.at[0]], nb, nsm)
                else:
                    pltpu.async_copy(hs.at[nw.at[2]], nb, nsm)
            pltpu.sync_copy(b, acc_sh.at[cw.at[2 * k + 1]], add=True)
            if k == 0 and w < NWIN - 1:
                pltpu.async_copy(
                    esd3d.at[wid, pl.ds((w + 1) * 2 * WIN, 2 * WIN)], nw, semi)

    plsc.subcore_barrier()

    @pl.when(s < NS - 1)
    def _():
        _drain_rows(acc_sh, out, c, base, RPT)

    @pl.when(s == NS - 1)
    def _():
        _drain_rows(acc_sh, out, c, base, RPT_LAST)


_agg_call = pl.kernel(
    _agg_body,
    out_type=jax.ShapeDtypeStruct((NC, N, D), jnp.float32),
    mesh=_sc_mesh,
    scratch_types=[
        pltpu.VMEM((2 * WIN, CH), jnp.int32),
        pltpu.VMEM((2 * WIN, CH), jnp.int32),
        pltpu.VMEM((CH, D), jnp.float32),
        pltpu.VMEM((CH, D), jnp.float32),
        pltpu.VMEM((CH, D), jnp.float32),
        pltpu.VMEM_SHARED((N, D), jnp.float32),
        pltpu.SemaphoreType.DMA,
        pltpu.SemaphoreType.DMA,
        pltpu.SemaphoreType.DMA,
        pltpu.SemaphoreType.DMA,
    ],
)


BLK = 2000
GRID = N // BLK


def _mm0_body(x_ref, w_ref, h_ref):
    h_ref[...] = jnp.dot(x_ref[...], w_ref[...], preferred_element_type=jnp.float32)


def _scale0_body(h_ref, p0_ref, p1_ref, hs_ref, dis_ref):
    deg = p0_ref[:, :1] + p1_ref[:, :1] + 1.0
    dis = lax.rsqrt(deg)
    hs_ref[...] = h_ref[...] * dis
    dis_ref[...] = dis


def _stage_mid_body(a0_ref, a1_ref, dis_ref, b_ref, g_ref, be_ref, m_ref,
                    v_ref, w_ref, hs_ref):
    dis = dis_ref[...]
    y = (a0_ref[...] + a1_ref[...]) * dis + b_ref[...]
    t = (y - m_ref[...]) * lax.rsqrt(v_ref[...] + EPS) * g_ref[...] + be_ref[...]
    t = jnp.maximum(t, 0.0)
    hs_ref[...] = jnp.dot(t, w_ref[...], preferred_element_type=jnp.float32) * dis


def _stage_fin_body(a0_ref, a1_ref, dis_ref, b2_ref, wm1_ref, bm1_ref,
                    wm2_ref, bm2_ref, out_ref):
    y = (a0_ref[...] + a1_ref[...]) * dis_ref[...] + b2_ref[...]
    z = jnp.dot(y, wm1_ref[...], preferred_element_type=jnp.float32) + bm1_ref[...]
    z = jnp.maximum(z, 0.0)
    out_ref[...] = jnp.dot(z, wm2_ref[...], preferred_element_type=jnp.float32) + bm2_ref[...]


def _row_spec(w):
    return pl.BlockSpec((BLK, w), lambda i: (i, 0))


def _full_spec(shape):
    return pl.BlockSpec(shape, lambda i: (0, 0))


_mm0_call = pl.pallas_call(
    _mm0_body,
    grid=(GRID,),
    in_specs=[_row_spec(D), _full_spec((D, D))],
    out_specs=_row_spec(D),
    out_shape=jax.ShapeDtypeStruct((N, D), jnp.float32),
)

_scale0_call = pl.pallas_call(
    _scale0_body,
    grid=(GRID,),
    in_specs=[_row_spec(D), _row_spec(DEGW), _row_spec(DEGW)],
    out_specs=[_row_spec(D), _row_spec(1)],
    out_shape=[
        jax.ShapeDtypeStruct((N, D), jnp.float32),
        jax.ShapeDtypeStruct((N, 1), jnp.float32),
    ],
)

_stage_mid_call = pl.pallas_call(
    _stage_mid_body,
    grid=(GRID,),
    in_specs=[_row_spec(D), _row_spec(D), _row_spec(1)]
    + [_full_spec((1, D))] * 5 + [_full_spec((D, D))],
    out_specs=_row_spec(D),
    out_shape=jax.ShapeDtypeStruct((N, D), jnp.float32),
)

_stage_fin_call = pl.pallas_call(
    _stage_fin_body,
    grid=(GRID,),
    in_specs=[_row_spec(D), _row_spec(D), _row_spec(1), _full_spec((1, D)),
              _full_spec((D, D)), _full_spec((1, D)), _full_spec((D, D)),
              _full_spec((1, D))],
    out_specs=_row_spec(D),
    out_shape=jax.ShapeDtypeStruct((N, D), jnp.float32),
)


def kernel(x, edge_index, W0, b0, g0, be0, m0, v0, W1, b1, g1, be1, m1, v1,
           W2, b2, Wm1, bm1, Wm2, bm2):
    ei = edge_index.astype(jnp.int32)
    src3d = ei[0].reshape(NW, NCH, CH)
    dst3d = ei[1].reshape(NW, NCH, CH)
    esd3d = jnp.stack([src3d, dst3d], axis=2).reshape(NW, 2 * NCH, CH)
    dst3dd = ei[1].reshape(NW, NCHD, CHD)

    ones16 = jnp.ones((CHD, DEGW), jnp.float32)
    zerosD = jnp.zeros((ZB, D), jnp.float32)

    degp = _deg_call(dst3dd, ones16, zerosD)
    h0 = _mm0_call(x, W0)
    hs0, dis = _scale0_call(h0, degp[0], degp[1])

    acc0 = _agg_call(hs0, esd3d, zerosD)
    hs1 = _stage_mid_call(acc0[0], acc0[1], dis, b0.reshape(1, D),
                          g0.reshape(1, D), be0.reshape(1, D),
                          m0.reshape(1, D), v0.reshape(1, D), W1)

    acc1 = _agg_call(hs1, esd3d, zerosD)
    hs2 = _stage_mid_call(acc1[0], acc1[1], dis, b1.reshape(1, D),
                          g1.reshape(1, D), be1.reshape(1, D),
                          m1.reshape(1, D), v1.reshape(1, D), W2)

    acc2 = _agg_call(hs2, esd3d, zerosD)
    out = _stage_fin_call(acc2[0], acc2[1], dis, b2.reshape(1, D),
                          Wm1, bm1.reshape(1, D), Wm2, bm2.reshape(1, D))
    return out
